# baseline (speedup 1.0000x reference)
import functools

import jax
import jax.numpy as jnp
from jax import lax
from jax.experimental import pallas as pl
from jax.experimental.pallas import tpu as pltpu
from jax.experimental.pallas import tpu_sc as plsc

B = 4096
D = 64

_NC, _NS = 2, 16
_NW = _NC * _NS
_BPW = B // _NW


def _gather_body(cw_hbm, xw_hbm, ctab_hbm, xtab_hbm, ce_out, xe_out,
                 idx_c, rows_c, idx_x, rows_x, sem_c, sem_x):
    wid = lax.axis_index("s") * _NC + lax.axis_index("c")
    base = wid * _BPW
    pltpu.sync_copy(cw_hbm.at[pl.ds(base, _BPW)], idx_c)
    pltpu.sync_copy(xw_hbm.at[pl.ds(base, _BPW)], idx_x)
    cpy_c = pltpu.async_copy(ctab_hbm.at[idx_c], rows_c, sem_c)
    cpy_x = pltpu.async_copy(xtab_hbm.at[idx_x], rows_x, sem_x)
    cpy_c.wait()
    cpy_x.wait()
    pltpu.sync_copy(rows_c, ce_out.at[pl.ds(base, _BPW)])
    pltpu.sync_copy(rows_x, xe_out.at[pl.ds(base, _BPW)])


@functools.lru_cache(maxsize=1)
def _make_gather():
    return pl.kernel(
        _gather_body,
        mesh=plsc.VectorSubcoreMesh(core_axis_name="c", subcore_axis_name="s"),
        out_type=[
            jax.ShapeDtypeStruct((B, D), jnp.float32),
            jax.ShapeDtypeStruct((B, D), jnp.float32),
        ],
        scratch_types=[
            pltpu.VMEM((_BPW,), jnp.int32),
            pltpu.VMEM((_BPW, D), jnp.float32),
            pltpu.VMEM((_BPW,), jnp.int32),
            pltpu.VMEM((_BPW, D), jnp.float32),
            pltpu.SemaphoreType.DMA,
            pltpu.SemaphoreType.DMA,
        ],
        compiler_params=pltpu.CompilerParams(use_tc_tiling_on_sc=False),
    )


_NBLK = 8
_RB = B // _NBLK


def _loss_body(ce_ref, cte_ref, cte_all_ref, out_ref, acc_ref):
    i = pl.program_id(0)

    @pl.when(i == 0)
    def _():
        acc_ref[0] = 0.0
        acc_ref[1] = 0.0

    ce = ce_ref[...]
    logits = lax.dot_general(ce, cte_all_ref[...], (((1,), (1,)), ((), ())),
                             preferred_element_type=jnp.float32)
    acc_ref[0] += jnp.sum(jnp.exp(logits))
    acc_ref[1] += jnp.sum(ce * cte_ref[...])

    @pl.when(i == _NBLK - 1)
    def _():
        out_ref[0] = jnp.log(acc_ref[0]) - acc_ref[1] / B


_loss = pl.pallas_call(
    _loss_body,
    grid=(_NBLK,),
    in_specs=[
        pl.BlockSpec((_RB, D), lambda i: (i, 0)),
        pl.BlockSpec((_RB, D), lambda i: (i, 0)),
        pl.BlockSpec((B, D), lambda i: (0, 0)),
    ],
    out_specs=pl.BlockSpec(memory_space=pltpu.SMEM),
    out_shape=jax.ShapeDtypeStruct((1,), jnp.float32),
    scratch_shapes=[pltpu.SMEM((2,), jnp.float32)],
)


def kernel(center_word, context_word, center_table, context_table):
    ce, cte = _make_gather()(center_word, context_word, center_table,
                             context_table)
    return _loss(ce, cte, cte)[0]


# speedup vs baseline: 1.3453x; 1.3453x over previous
import functools

import jax
import jax.numpy as jnp
from jax import lax
from jax.experimental import pallas as pl
from jax.experimental.pallas import tpu as pltpu
from jax.experimental.pallas import tpu_sc as plsc

B = 4096
D = 64

_NC, _NS = 2, 16
_NW = _NC * _NS
_BPW = B // _NW
_LANES = 16


def _gather_body(cw_hbm, xw_hbm, ctab_hbm, xtab_hbm, ce_out, xe_out,
                 idx_c, idx_x, sem_c, sem_x):
    w = lax.axis_index("s") * _NC + lax.axis_index("c")
    base = w * _BPW
    pltpu.sync_copy(cw_hbm.at[pl.ds(base, _BPW)], idx_c)
    pltpu.sync_copy(xw_hbm.at[pl.ds(base, _BPW)], idx_x)

    def fire(tab_hbm, out_hbm, idx_v, sem):
        def chunk(c, carry):
            vec = idx_v[pl.ds(c * _LANES, _LANES)]
            for lane in range(_LANES):
                r = vec[lane]
                pltpu.async_copy(
                    tab_hbm.at[pl.ds(r, 1), :],
                    out_hbm.at[pl.ds(base + c * _LANES + lane, 1), :],
                    sem,
                )
            return carry

        lax.fori_loop(0, _BPW // _LANES, chunk, 0)

    fire(ctab_hbm, ce_out, idx_c, sem_c)
    fire(xtab_hbm, xe_out, idx_x, sem_x)
    pltpu.make_async_copy(
        ctab_hbm.at[pl.ds(0, _BPW), :], ce_out.at[pl.ds(base, _BPW), :], sem_c
    ).wait()
    pltpu.make_async_copy(
        xtab_hbm.at[pl.ds(0, _BPW), :], xe_out.at[pl.ds(base, _BPW), :], sem_x
    ).wait()


@functools.lru_cache(maxsize=1)
def _make_gather():
    return pl.kernel(
        _gather_body,
        mesh=plsc.VectorSubcoreMesh(core_axis_name="c", subcore_axis_name="s"),
        out_type=[
            jax.ShapeDtypeStruct((B, D), jnp.float32),
            jax.ShapeDtypeStruct((B, D), jnp.float32),
        ],
        scratch_types=[
            pltpu.VMEM((_BPW,), jnp.int32),
            pltpu.VMEM((_BPW,), jnp.int32),
            pltpu.SemaphoreType.DMA,
            pltpu.SemaphoreType.DMA,
        ],
    )


_NBLK = 8
_RB = B // _NBLK


def _loss_body(ce_ref, cte_ref, cte_all_ref, out_ref, acc_ref):
    i = pl.program_id(0)

    @pl.when(i == 0)
    def _():
        acc_ref[0] = 0.0
        acc_ref[1] = 0.0

    ce = ce_ref[...]
    logits = lax.dot_general(ce, cte_all_ref[...], (((1,), (1,)), ((), ())),
                             preferred_element_type=jnp.float32)
    acc_ref[0] += jnp.sum(jnp.exp(logits))
    acc_ref[1] += jnp.sum(ce * cte_ref[...])

    @pl.when(i == _NBLK - 1)
    def _():
        out_ref[0] = jnp.log(acc_ref[0]) - acc_ref[1] / B


_loss = pl.pallas_call(
    _loss_body,
    grid=(_NBLK,),
    in_specs=[
        pl.BlockSpec((_RB, D), lambda i: (i, 0)),
        pl.BlockSpec((_RB, D), lambda i: (i, 0)),
        pl.BlockSpec((B, D), lambda i: (0, 0)),
    ],
    out_specs=pl.BlockSpec(memory_space=pltpu.SMEM),
    out_shape=jax.ShapeDtypeStruct((1,), jnp.float32),
    scratch_shapes=[pltpu.SMEM((2,), jnp.float32)],
)


def kernel(center_word, context_word, center_table, context_table):
    ce, cte = _make_gather()(center_word, context_word, center_table,
                             context_table)
    return _loss(ce, cte, cte)[0]


# speedup vs baseline: 6.9348x; 5.1550x over previous
import functools

import jax
import jax.numpy as jnp
from jax import lax
from jax.experimental import pallas as pl
from jax.experimental.pallas import tpu as pltpu
from jax.experimental.pallas import tpu_sc as plsc

B = 4096
D = 64

_NC, _NS = 2, 16
_NW = _NC * _NS
_BPW = B // _NW
_L = 16
_NBUF = 4


def _dyn_idx(idx_ref, i):
    v = plsc.load_gather(idx_ref, [jnp.full((_L,), i, jnp.int32)])
    return v[0]


def _gather_body(cw_hbm, xw_hbm, ctabT_hbm, xtabT_hbm, ce_out, xe_out,
                 idx_c, idx_x, slabs, cols, sems):
    w = lax.axis_index("s") * _NC + lax.axis_index("c")
    base = w * _BPW
    pltpu.sync_copy(cw_hbm.at[pl.ds(base, _BPW)], idx_c)
    pltpu.sync_copy(xw_hbm.at[pl.ds(base, _BPW)], idx_x)

    def run_table(tabT_hbm, idx_v, out_hbm):
        def fire(i, b):
            r = _dyn_idx(idx_v, i)
            off = pl.multiple_of((r >> 7) * 128, 128)
            pltpu.async_copy(
                tabT_hbm.at[:, pl.ds(off, 128)], slabs.at[b], sems.at[b]
            )

        def extract(i, b):
            r = _dyn_idx(idx_v, i)
            lane = jnp.full((_L,), r & 127, jnp.int32)
            for q in range(D // _L):
                drows = jax.lax.iota(jnp.int32, _L) + (q * _L)
                vals = plsc.load_gather(slabs.at[b], [drows, lane])
                plsc.store_scatter(
                    cols,
                    [jnp.full((_L,), i, jnp.int32), drows],
                    vals,
                )

        for b in range(_NBUF):
            fire(b, b)

        def group(g, carry):
            for b in range(_NBUF):
                i = g * _NBUF + b
                pltpu.make_async_copy(
                    tabT_hbm.at[:, pl.ds(0, 128)], slabs.at[b], sems.at[b]
                ).wait()
                extract(i, b)

                @pl.when(g < (_BPW // _NBUF) - 1)
                def _():
                    fire(i + _NBUF, b)
            return carry

        lax.fori_loop(0, _BPW // _NBUF, group, 0)
        pltpu.sync_copy(cols, out_hbm.at[pl.ds(base, _BPW)])

    run_table(ctabT_hbm, idx_c, ce_out)
    run_table(xtabT_hbm, idx_x, xe_out)


@functools.lru_cache(maxsize=1)
def _make_gather():
    return pl.kernel(
        _gather_body,
        mesh=plsc.VectorSubcoreMesh(core_axis_name="c", subcore_axis_name="s"),
        out_type=[
            jax.ShapeDtypeStruct((B, D), jnp.float32),
            jax.ShapeDtypeStruct((B, D), jnp.float32),
        ],
        scratch_types=[
            pltpu.VMEM((_BPW,), jnp.int32),
            pltpu.VMEM((_BPW,), jnp.int32),
            pltpu.VMEM((_NBUF, D, 128), jnp.float32),
            pltpu.VMEM((_BPW, D), jnp.float32),
            pltpu.SemaphoreType.DMA((_NBUF,)),
        ],
        compiler_params=pltpu.CompilerParams(needs_layout_passes=False),
    )


_NBLK = 8
_RB = B // _NBLK


def _loss_body(ce_ref, cte_ref, cte_all_ref, out_ref, acc_ref):
    i = pl.program_id(0)

    @pl.when(i == 0)
    def _():
        acc_ref[0] = 0.0
        acc_ref[1] = 0.0

    ce = ce_ref[...]
    logits = lax.dot_general(ce, cte_all_ref[...], (((1,), (1,)), ((), ())),
                             preferred_element_type=jnp.float32)
    acc_ref[0] += jnp.sum(jnp.exp(logits))
    acc_ref[1] += jnp.sum(ce * cte_ref[...])

    @pl.when(i == _NBLK - 1)
    def _():
        out_ref[0] = jnp.log(acc_ref[0]) - acc_ref[1] / B


_loss = pl.pallas_call(
    _loss_body,
    grid=(_NBLK,),
    in_specs=[
        pl.BlockSpec((_RB, D), lambda i: (i, 0)),
        pl.BlockSpec((_RB, D), lambda i: (i, 0)),
        pl.BlockSpec((B, D), lambda i: (0, 0)),
    ],
    out_specs=pl.BlockSpec(memory_space=pltpu.SMEM),
    out_shape=jax.ShapeDtypeStruct((1,), jnp.float32),
    scratch_shapes=[pltpu.SMEM((2,), jnp.float32)],
)


def kernel(center_word, context_word, center_table, context_table):
    ce, cte = _make_gather()(center_word, context_word, center_table.T,
                             context_table.T)
    return _loss(ce, cte, cte)[0]


# speedup vs baseline: 7.8700x; 1.1349x over previous
import functools

import jax
import jax.numpy as jnp
from jax import lax
from jax.experimental import pallas as pl
from jax.experimental.pallas import tpu as pltpu
from jax.experimental.pallas import tpu_sc as plsc

B = 4096
D = 64

_NC, _NS = 2, 16
_NW = _NC * _NS
_BPW = B // _NW
_L = 16
_NBUF = 8


def _dyn_idx(idx_ref, i):
    v = plsc.load_gather(idx_ref, [jnp.full((_L,), i, jnp.int32)])
    return v[0]


def _gather_body(cw_hbm, xw_hbm, ctabT_hbm, xtabT_hbm, ce_out, xe_out,
                 idx_c, idx_x, slabs, cols, sems):
    w = lax.axis_index("s") * _NC + lax.axis_index("c")
    base = w * _BPW
    pltpu.sync_copy(cw_hbm.at[pl.ds(base, _BPW)], idx_c)
    pltpu.sync_copy(xw_hbm.at[pl.ds(base, _BPW)], idx_x)

    tabs = (ctabT_hbm, xtabT_hbm)
    idxs = (idx_c, idx_x)
    outs = (ce_out, xe_out)
    half = _NBUF // 2

    def fire(t, i, b):
        r = _dyn_idx(idxs[t], i)
        off = pl.multiple_of((r >> 7) * 128, 128)
        bb = t * half + b
        pltpu.async_copy(
            tabs[t].at[:, pl.ds(off, 128)], slabs.at[bb], sems.at[bb]
        )

    def extract(t, i, b):
        r = _dyn_idx(idxs[t], i)
        bb = t * half + b
        lane = jnp.full((_L,), r & 127, jnp.int32)
        for q in range(D // _L):
            drows = jax.lax.iota(jnp.int32, _L) + (q * _L)
            vals = plsc.load_gather(slabs.at[bb], [drows, lane])
            plsc.store_scatter(
                cols.at[t],
                [jnp.full((_L,), i, jnp.int32), drows],
                vals,
            )

    for b in range(half):
        fire(0, b, b)
        fire(1, b, b)

    def group(g, carry):
        for b in range(half):
            i = g * half + b
            for t in range(2):
                bb = t * half + b
                pltpu.make_async_copy(
                    tabs[t].at[:, pl.ds(0, 128)], slabs.at[bb], sems.at[bb]
                ).wait()
                extract(t, i, b)

                @pl.when(g < (_BPW // half) - 1)
                def _():
                    fire(t, i + half, b)
        return carry

    lax.fori_loop(0, _BPW // half, group, 0)
    pltpu.sync_copy(cols.at[0], ce_out.at[pl.ds(base, _BPW)])
    pltpu.sync_copy(cols.at[1], xe_out.at[pl.ds(base, _BPW)])


@functools.lru_cache(maxsize=1)
def _make_gather():
    return pl.kernel(
        _gather_body,
        mesh=plsc.VectorSubcoreMesh(core_axis_name="c", subcore_axis_name="s"),
        out_type=[
            jax.ShapeDtypeStruct((B, D), jnp.float32),
            jax.ShapeDtypeStruct((B, D), jnp.float32),
        ],
        scratch_types=[
            pltpu.VMEM((_BPW,), jnp.int32),
            pltpu.VMEM((_BPW,), jnp.int32),
            pltpu.VMEM((_NBUF, D, 128), jnp.float32),
            pltpu.VMEM((2, _BPW, D), jnp.float32),
            pltpu.SemaphoreType.DMA((_NBUF,)),
        ],
        compiler_params=pltpu.CompilerParams(needs_layout_passes=False),
    )


_NBLK = 8
_RB = B // _NBLK


def _loss_body(ce_ref, cte_ref, cte_all_ref, out_ref, acc_ref):
    i = pl.program_id(0)

    @pl.when(i == 0)
    def _():
        acc_ref[0] = 0.0
        acc_ref[1] = 0.0

    ce = ce_ref[...]
    logits = lax.dot_general(ce, cte_all_ref[...], (((1,), (1,)), ((), ())),
                             preferred_element_type=jnp.float32)
    acc_ref[0] += jnp.sum(jnp.exp(logits))
    acc_ref[1] += jnp.sum(ce * cte_ref[...])

    @pl.when(i == _NBLK - 1)
    def _():
        out_ref[0] = jnp.log(acc_ref[0]) - acc_ref[1] / B


_loss = pl.pallas_call(
    _loss_body,
    grid=(_NBLK,),
    in_specs=[
        pl.BlockSpec((_RB, D), lambda i: (i, 0)),
        pl.BlockSpec((_RB, D), lambda i: (i, 0)),
        pl.BlockSpec((B, D), lambda i: (0, 0)),
    ],
    out_specs=pl.BlockSpec(memory_space=pltpu.SMEM),
    out_shape=jax.ShapeDtypeStruct((1,), jnp.float32),
    scratch_shapes=[pltpu.SMEM((2,), jnp.float32)],
)


def kernel(center_word, context_word, center_table, context_table):
    ce, cte = _make_gather()(center_word, context_word, center_table.T,
                             context_table.T)
    return _loss(ce, cte, cte)[0]


# speedup vs baseline: 7.9010x; 1.0039x over previous
import functools

import jax
import jax.numpy as jnp
from jax import lax
from jax.experimental import pallas as pl
from jax.experimental.pallas import tpu as pltpu
from jax.experimental.pallas import tpu_sc as plsc

B = 4096
D = 64

_NC, _NS = 2, 16
_NW = _NC * _NS
_BPW = B // _NW
_L = 16
_NBUF = 8


def _dyn_idx(idx_ref, i):
    v = plsc.load_gather(idx_ref, [jnp.full((_L,), i, jnp.int32)])
    return v[0]


def _gather_body(cw_hbm, xw_hbm, ctabT_hbm, xtabT_hbm, ce_out, xe_out,
                 idx_c, idx_x, slabs, cols, sems):
    w = lax.axis_index("s") * _NC + lax.axis_index("c")
    base = w * _BPW
    pltpu.sync_copy(cw_hbm.at[pl.ds(base, _BPW)], idx_c)
    pltpu.sync_copy(xw_hbm.at[pl.ds(base, _BPW)], idx_x)

    tabs = (ctabT_hbm, xtabT_hbm)
    idxs = (idx_c, idx_x)
    outs = (ce_out, xe_out)
    half = _NBUF // 2

    def fire(t, i, b):
        r = _dyn_idx(idxs[t], i)
        off = pl.multiple_of((r >> 7) * 128, 128)
        bb = t * half + b
        pltpu.async_copy(
            tabs[t].at[:, pl.ds(off, 128)], slabs.at[bb], sems.at[bb]
        )

    def extract(t, i, b):
        r = _dyn_idx(idxs[t], i)
        bb = t * half + b
        lane = jnp.full((_L,), r & 127, jnp.int32)
        for q in range(D // _L):
            drows = jax.lax.iota(jnp.int32, _L) + (q * _L)
            vals = plsc.load_gather(slabs.at[bb], [drows, lane])
            plsc.store_scatter(
                cols.at[t],
                [jnp.full((_L,), i, jnp.int32), drows],
                vals,
            )

    for b in range(half):
        fire(0, b, b)
        fire(1, b, b)

    def group(g, carry):
        for b in range(half):
            i = g * half + b
            for t in range(2):
                bb = t * half + b
                pltpu.make_async_copy(
                    tabs[t].at[:, pl.ds(0, 128)], slabs.at[bb], sems.at[bb]
                ).wait()
                extract(t, i, b)

                @pl.when(g < (_BPW // half) - 1)
                def _():
                    fire(t, i + half, b)
        return carry

    lax.fori_loop(0, _BPW // half, group, 0)
    pltpu.sync_copy(cols.at[0], ce_out.at[pl.ds(base, _BPW)])
    pltpu.sync_copy(cols.at[1], xe_out.at[pl.ds(base, _BPW)])


@functools.lru_cache(maxsize=1)
def _make_gather():
    return pl.kernel(
        _gather_body,
        mesh=plsc.VectorSubcoreMesh(core_axis_name="c", subcore_axis_name="s"),
        out_type=[
            jax.ShapeDtypeStruct((B, D), jnp.float32),
            jax.ShapeDtypeStruct((B, D), jnp.float32),
        ],
        scratch_types=[
            pltpu.VMEM((_BPW,), jnp.int32),
            pltpu.VMEM((_BPW,), jnp.int32),
            pltpu.VMEM((_NBUF, D, 128), jnp.float32),
            pltpu.VMEM((2, _BPW, D), jnp.float32),
            pltpu.SemaphoreType.DMA((_NBUF,)),
        ],
        compiler_params=pltpu.CompilerParams(needs_layout_passes=False),
    )


_NBLK = 4
_RB = B // _NBLK


def _loss_body(ce_ref, cte_ref, cte_all_ref, out_ref, acc_ref):
    i = pl.program_id(0)

    @pl.when(i == 0)
    def _():
        acc_ref[0] = 0.0
        acc_ref[1] = 0.0

    ce = ce_ref[...]
    logits = lax.dot_general(ce, cte_all_ref[...], (((1,), (1,)), ((), ())),
                             preferred_element_type=jnp.float32)
    acc_ref[0] += jnp.sum(jnp.exp(logits))
    acc_ref[1] += jnp.sum(ce * cte_ref[...])

    @pl.when(i == _NBLK - 1)
    def _():
        out_ref[0] = jnp.log(acc_ref[0]) - acc_ref[1] / B


_loss = pl.pallas_call(
    _loss_body,
    grid=(_NBLK,),
    in_specs=[
        pl.BlockSpec((_RB, D), lambda i: (i, 0)),
        pl.BlockSpec((_RB, D), lambda i: (i, 0)),
        pl.BlockSpec((B, D), lambda i: (0, 0)),
    ],
    out_specs=pl.BlockSpec(memory_space=pltpu.SMEM),
    out_shape=jax.ShapeDtypeStruct((1,), jnp.float32),
    scratch_shapes=[pltpu.SMEM((2,), jnp.float32)],
)


def kernel(center_word, context_word, center_table, context_table):
    ce, cte = _make_gather()(center_word, context_word, center_table.T,
                             context_table.T)
    return _loss(ce, cte, cte)[0]


# speedup vs baseline: 7.9369x; 1.0045x over previous
import functools

import jax
import jax.numpy as jnp
from jax import lax
from jax.experimental import pallas as pl
from jax.experimental.pallas import tpu as pltpu
from jax.experimental.pallas import tpu_sc as plsc

B = 4096
D = 64

_NC, _NS = 2, 16
_NW = _NC * _NS
_BPW = B // _NW
_L = 16
_NBUF = 8


def _dyn_idx(idx_ref, i):
    v = plsc.load_gather(idx_ref, [jnp.full((_L,), i, jnp.int32)])
    return v[0]


def _gather_body(cw_hbm, xw_hbm, ctabT_hbm, xtabT_hbm, ce_out, xe_out,
                 idx_c, idx_x, slabs, cols, sems):
    w = lax.axis_index("s") * _NC + lax.axis_index("c")
    base = w * _BPW
    pltpu.sync_copy(cw_hbm.at[pl.ds(base, _BPW)], idx_c)
    pltpu.sync_copy(xw_hbm.at[pl.ds(base, _BPW)], idx_x)

    tabs = (ctabT_hbm, xtabT_hbm)
    idxs = (idx_c, idx_x)
    outs = (ce_out, xe_out)
    half = _NBUF // 2

    def fire(t, i, b):
        r = _dyn_idx(idxs[t], i)
        off = pl.multiple_of((r >> 7) * 128, 128)
        bb = t * half + b
        pltpu.async_copy(
            tabs[t].at[:, pl.ds(off, 128)], slabs.at[bb], sems.at[bb]
        )

    def extract(t, i, b):
        r = _dyn_idx(idxs[t], i)
        bb = t * half + b
        lane = jnp.full((_L,), r & 127, jnp.int32)
        for q in range(D // _L):
            drows = jax.lax.iota(jnp.int32, _L) + (q * _L)
            vals = plsc.load_gather(slabs.at[bb], [drows, lane])
            plsc.store_scatter(
                cols.at[t],
                [jnp.full((_L,), i, jnp.int32), drows],
                vals,
            )

    for b in range(half):
        fire(0, b, b)
        fire(1, b, b)

    def group(g, carry):
        for b in range(half):
            i = g * half + b
            for t in range(2):
                bb = t * half + b
                pltpu.make_async_copy(
                    tabs[t].at[:, pl.ds(0, 128)], slabs.at[bb], sems.at[bb]
                ).wait()
                extract(t, i, b)

                @pl.when(g < (_BPW // half) - 1)
                def _():
                    fire(t, i + half, b)
        return carry

    lax.fori_loop(0, _BPW // half, group, 0)
    pltpu.sync_copy(cols.at[0], ce_out.at[pl.ds(base, _BPW)])
    pltpu.sync_copy(cols.at[1], xe_out.at[pl.ds(base, _BPW)])


@functools.lru_cache(maxsize=1)
def _make_gather():
    return pl.kernel(
        _gather_body,
        mesh=plsc.VectorSubcoreMesh(core_axis_name="c", subcore_axis_name="s"),
        out_type=[
            jax.ShapeDtypeStruct((B, D), jnp.float32),
            jax.ShapeDtypeStruct((B, D), jnp.float32),
        ],
        scratch_types=[
            pltpu.VMEM((_BPW,), jnp.int32),
            pltpu.VMEM((_BPW,), jnp.int32),
            pltpu.VMEM((_NBUF, D, 128), jnp.float32),
            pltpu.VMEM((2, _BPW, D), jnp.float32),
            pltpu.SemaphoreType.DMA((_NBUF,)),
        ],
        compiler_params=pltpu.CompilerParams(needs_layout_passes=False),
    )


_NBLK = 4
_RB = B // _NBLK


def _loss_body(ce_ref, cte_ref, cte_all_ref, out_ref, acc_ref):
    i = pl.program_id(0)

    @pl.when(i == 0)
    def _():
        acc_ref[0] = 0.0
        acc_ref[1] = 0.0

    ce = ce_ref[...]
    logits = lax.dot_general(ce.astype(jnp.bfloat16),
                             cte_all_ref[...].astype(jnp.bfloat16),
                             (((1,), (1,)), ((), ())),
                             preferred_element_type=jnp.float32)
    acc_ref[0] += jnp.sum(jnp.exp(logits))
    acc_ref[1] += jnp.sum(ce * cte_ref[...])

    @pl.when(i == _NBLK - 1)
    def _():
        out_ref[0] = jnp.log(acc_ref[0]) - acc_ref[1] / B


_loss = pl.pallas_call(
    _loss_body,
    grid=(_NBLK,),
    in_specs=[
        pl.BlockSpec((_RB, D), lambda i: (i, 0)),
        pl.BlockSpec((_RB, D), lambda i: (i, 0)),
        pl.BlockSpec((B, D), lambda i: (0, 0)),
    ],
    out_specs=pl.BlockSpec(memory_space=pltpu.SMEM),
    out_shape=jax.ShapeDtypeStruct((1,), jnp.float32),
    scratch_shapes=[pltpu.SMEM((2,), jnp.float32)],
)


def kernel(center_word, context_word, center_table, context_table):
    ce, cte = _make_gather()(center_word, context_word, center_table.T,
                             context_table.T)
    return _loss(ce, cte, cte)[0]


# speedup vs baseline: 7.9479x; 1.0014x over previous
import functools

import jax
import jax.numpy as jnp
from jax import lax
from jax.experimental import pallas as pl
from jax.experimental.pallas import tpu as pltpu
from jax.experimental.pallas import tpu_sc as plsc

B = 4096
D = 64

_NC, _NS = 2, 16
_NW = _NC * _NS
_BPW = B // _NW
_L = 16
_NBUF = 8


def _dyn_idx(idx_ref, i):
    v = plsc.load_gather(idx_ref, [jnp.full((_L,), i, jnp.int32)])
    return v[0]


def _gather_body(cw_hbm, xw_hbm, ctabT_hbm, xtabT_hbm, ce_out, xe_out,
                 idx_c, idx_x, slabs, cols, sems):
    w = lax.axis_index("s") * _NC + lax.axis_index("c")
    base = w * _BPW
    pltpu.sync_copy(cw_hbm.at[pl.ds(base, _BPW)], idx_c)
    pltpu.sync_copy(xw_hbm.at[pl.ds(base, _BPW)], idx_x)

    tabs = (ctabT_hbm, xtabT_hbm)
    idxs = (idx_c, idx_x)
    outs = (ce_out, xe_out)
    half = _NBUF // 2

    def fire(t, i, b):
        r = _dyn_idx(idxs[t], i)
        off = pl.multiple_of((r >> 7) * 128, 128)
        bb = t * half + b
        pltpu.async_copy(
            tabs[t].at[:, pl.ds(off, 128)], slabs.at[bb], sems.at[bb]
        )

    def extract(t, i, b):
        r = _dyn_idx(idxs[t], i)
        bb = t * half + b
        lane = jnp.full((_L,), r & 127, jnp.int32)
        for q in range(D // _L):
            drows = jax.lax.iota(jnp.int32, _L) + (q * _L)
            vals = plsc.load_gather(slabs.at[bb], [drows, lane])
            plsc.store_scatter(
                cols.at[t],
                [jnp.full((_L,), i, jnp.int32), drows],
                vals,
            )

    for b in range(half):
        fire(0, b, b)
        fire(1, b, b)

    def group(g, carry):
        for b in range(half):
            i = g * half + b
            for t in range(2):
                bb = t * half + b
                pltpu.make_async_copy(
                    tabs[t].at[:, pl.ds(0, 128)], slabs.at[bb], sems.at[bb]
                ).wait()
                extract(t, i, b)

                @pl.when(g < (_BPW // half) - 1)
                def _():
                    fire(t, i + half, b)
        return carry

    lax.fori_loop(0, _BPW // half, group, 0)
    pltpu.sync_copy(cols.at[0], ce_out.at[pl.ds(base, _BPW)])
    pltpu.sync_copy(cols.at[1], xe_out.at[pl.ds(base, _BPW)])


@functools.lru_cache(maxsize=1)
def _make_gather():
    return pl.kernel(
        _gather_body,
        mesh=plsc.VectorSubcoreMesh(core_axis_name="c", subcore_axis_name="s"),
        out_type=[
            jax.ShapeDtypeStruct((B, D), jnp.float32),
            jax.ShapeDtypeStruct((B, D), jnp.float32),
        ],
        scratch_types=[
            pltpu.VMEM((_BPW,), jnp.int32),
            pltpu.VMEM((_BPW,), jnp.int32),
            pltpu.VMEM((_NBUF, D, 128), jnp.float32),
            pltpu.VMEM((2, _BPW, D), jnp.float32),
            pltpu.SemaphoreType.DMA((_NBUF,)),
        ],
        compiler_params=pltpu.CompilerParams(needs_layout_passes=False),
    )


_NBLK = 2
_RB = B // _NBLK


def _loss_body(ce_ref, cte_ref, cte_all_ref, out_ref, acc_ref):
    i = pl.program_id(0)

    @pl.when(i == 0)
    def _():
        acc_ref[0] = 0.0
        acc_ref[1] = 0.0

    ce = ce_ref[...]
    logits = lax.dot_general(ce.astype(jnp.bfloat16),
                             cte_all_ref[...].astype(jnp.bfloat16),
                             (((1,), (1,)), ((), ())),
                             preferred_element_type=jnp.float32)
    acc_ref[0] += jnp.sum(jnp.exp(logits))
    acc_ref[1] += jnp.sum(ce * cte_ref[...])

    @pl.when(i == _NBLK - 1)
    def _():
        out_ref[0] = jnp.log(acc_ref[0]) - acc_ref[1] / B


_loss = pl.pallas_call(
    _loss_body,
    grid=(_NBLK,),
    in_specs=[
        pl.BlockSpec((_RB, D), lambda i: (i, 0)),
        pl.BlockSpec((_RB, D), lambda i: (i, 0)),
        pl.BlockSpec((B, D), lambda i: (0, 0)),
    ],
    out_specs=pl.BlockSpec(memory_space=pltpu.SMEM),
    out_shape=jax.ShapeDtypeStruct((1,), jnp.float32),
    scratch_shapes=[pltpu.SMEM((2,), jnp.float32)],
)


def kernel(center_word, context_word, center_table, context_table):
    ce, cte = _make_gather()(center_word, context_word, center_table.T,
                             context_table.T)
    return _loss(ce, cte, cte)[0]
